# baseline (speedup 1.0000x reference)
import functools

import jax
import jax.numpy as jnp
from jax import lax
from jax.experimental import pallas as pl
from jax.experimental.pallas import tpu as pltpu
from jax.experimental.pallas import tpu_sc as plsc

_NUM_CORES = 2
_NUM_SUBCORES = 16
_NUM_WORKERS = _NUM_CORES * _NUM_SUBCORES

_CH = 128
_NBUF = 4


@functools.partial(jax.jit, static_argnames=())
def _embedding_lookup(idx2d, table):
    n_blocks, ch = idx2d.shape
    v, d = table.shape
    n = n_blocks * ch
    blocks_per_w = n_blocks // _NUM_WORKERS
    rows_per_w = blocks_per_w * ch

    mesh = plsc.VectorSubcoreMesh(core_axis_name="c", subcore_axis_name="s")

    @functools.partial(
        pl.kernel,
        out_type=jax.ShapeDtypeStruct((n, d), jnp.float32),
        mesh=mesh,
        scratch_types=[
            pltpu.VMEM((blocks_per_w, ch), jnp.int32),
            pltpu.VMEM((_NBUF, ch, d), jnp.float32),
            pltpu.SemaphoreType.DMA((_NBUF,)),
        ],
        compiler_params=pltpu.CompilerParams(use_tc_tiling_on_sc=False),
    )
    def emb(idx_hbm, table_hbm, out_hbm, idx_v, rows_v, gsem):
        wid = lax.axis_index("s") * _NUM_CORES + lax.axis_index("c")
        blk0 = wid * blocks_per_w
        base = wid * rows_per_w

        pltpu.sync_copy(idx_hbm.at[pl.ds(blk0, blocks_per_w)], idx_v)

        def start_gather(g, b):
            pltpu.async_copy(table_hbm.at[idx_v.at[g]], rows_v.at[b], gsem.at[b])

        def wait_gather(b):
            pltpu.make_async_copy(
                table_hbm.at[pl.ds(0, ch)], rows_v.at[b], gsem.at[b]
            ).wait()

        for b in range(_NBUF):
            start_gather(b, b)

        @pl.loop(0, blocks_per_w // _NBUF - 1)
        def _(it):
            g0 = it * _NBUF
            for b in range(_NBUF):
                wait_gather(b)
                pltpu.sync_copy(
                    rows_v.at[b], out_hbm.at[pl.ds(base + (g0 + b) * ch, ch)]
                )
                start_gather(g0 + b + _NBUF, b)

        g0 = blocks_per_w - _NBUF
        for b in range(_NBUF):
            wait_gather(b)
            pltpu.sync_copy(
                rows_v.at[b], out_hbm.at[pl.ds(base + (g0 + b) * ch, ch)]
            )

    return emb(idx2d, table)


def kernel(inputs, table):
    batch, seq = inputs.shape
    _, d = table.shape
    n = batch * seq
    idx2d = inputs.reshape(n // _CH, _CH).astype(jnp.int32)
    out = _embedding_lookup(idx2d, table)
    return out.reshape(batch, seq, d)
